# baseline (speedup 1.0000x reference)
import functools

import jax
import jax.numpy as jnp
from jax.experimental import pallas as pl
from jax.experimental.pallas import tpu as pltpu
from jax.experimental.pallas import tpu_sc as plsc


def _elu(x):
    return jnp.where(x > 0, x, jnp.exp(jnp.minimum(x, 0.0)) - 1.0)


def _gat_block(feats, asn, neg_mask, H, C):
    N = feats.shape[0]
    acc = jnp.zeros((N, C), jnp.float32)
    inv_h = 1.0 / H
    feats_b = feats.astype(jnp.bfloat16)
    esen = jnp.dot(feats, asn, preferred_element_type=jnp.float32)
    enT = jnp.transpose(esen[:, H:])
    for h in range(H):
        t = esen[:, h:h + 1] + enT[h:h + 1, :]
        p = jnp.exp2(jnp.maximum(t, 0.2 * t) + neg_mask)
        pb = p.astype(jnp.bfloat16)
        s = jnp.sum(p, axis=1, keepdims=True)
        acc = acc + jnp.dot(pb, feats_b[:, h * C:(h + 1) * C],
                            preferred_element_type=jnp.float32) * (inv_h / s)
    return acc


def _mask_body(a_ref, m_ref):
    Nn = a_ref.shape[1]
    a = a_ref[0]
    ri = jax.lax.broadcasted_iota(jnp.int32, (Nn, Nn), 0)
    ci = jax.lax.broadcasted_iota(jnp.int32, (Nn, Nn), 1)
    edge = jnp.logical_or(a > 0.5, ri == ci)
    m_ref[0] = jnp.where(edge, 0.0, -1e9).astype(jnp.bfloat16)


def _tc_body(scale_ref, xu_ref, m_ref, down_ref, orig_ref,
             wup_ref, asn_up_ref, wend_ref, asn_end_ref, out_ref):
    F = xu_ref.shape[2]
    H = asn_up_ref.shape[1] // 2
    C = wup_ref.shape[1] // H

    neg_mask = m_ref[0].astype(jnp.float32)

    feats1 = jnp.dot(xu_ref[0], wup_ref[...],
                     preferred_element_type=jnp.float32)
    x1 = _elu(_gat_block(feats1, asn_up_ref[...], neg_mask, H, C))
    x1 = x1 + down_ref[0]

    feats2 = (jnp.dot(x1, wend_ref[:F, :], preferred_element_type=jnp.float32)
              + jnp.dot(orig_ref[0], wend_ref[F:, :],
                        preferred_element_type=jnp.float32))
    out_ref[0] = _elu(_gat_block(feats2, asn_end_ref[...], neg_mask,
                                 H, C)) * scale_ref[0]


def _sc_unpool(X, idx0, Nn):
    B, No, F = X.shape
    NC, NS = 2, 16
    bpc = B // NC
    rows_w = No // NS
    zrows = bpc * Nn // NS
    mesh = plsc.VectorSubcoreMesh(core_axis_name="c", subcore_axis_name="s")

    @functools.partial(
        pl.kernel, mesh=mesh,
        out_type=jax.ShapeDtypeStruct((B * Nn, F), jnp.float32),
        scratch_types=[
            pltpu.VMEM((rows_w, F), jnp.float32),
            pltpu.VMEM((rows_w,), jnp.int32),
            pltpu.VMEM((zrows, F), jnp.float32),
            pltpu.VMEM_SHARED((bpc * Nn, F), jnp.float32),
        ])
    def k(x_hbm, idx_hbm, out_hbm, rows_v, idx_v, zbuf, acc_sh):
        c = jax.lax.axis_index("c")
        s = jax.lax.axis_index("s")

        def zero_row(r, carry):
            for kk in range(F // 16):
                zbuf[r, pl.ds(kk * 16, 16)] = jnp.zeros((16,), jnp.float32)
            return carry

        jax.lax.fori_loop(0, zrows, zero_row, 0)
        pltpu.sync_copy(zbuf, acc_sh.at[pl.ds(s * zrows, zrows)])
        plsc.subcore_barrier()

        for lb in range(bpc):
            b = c * bpc + lb
            pltpu.sync_copy(idx_hbm.at[b, pl.ds(s * rows_w, rows_w)], idx_v)
            for kk in range(rows_w // 16):
                idx_v[pl.ds(kk * 16, 16)] = (idx_v[pl.ds(kk * 16, 16)]
                                             + lb * Nn)
            pltpu.sync_copy(x_hbm.at[b, pl.ds(s * rows_w, rows_w)], rows_v)
            pltpu.sync_copy(rows_v, acc_sh.at[idx_v], add=True)
        plsc.subcore_barrier()

        base = c * (bpc * Nn) + s * zrows
        pltpu.sync_copy(acc_sh.at[pl.ds(s * zrows, zrows)],
                        out_hbm.at[pl.ds(base, zrows)])

    return k(X, idx0)


def kernel(X, orig_X, l_n, idx0, A0, down0, action, W_up, a_s_up, a_n_up,
           W_end, a_s_end, a_n_end):
    B, No, F = X.shape
    Nn = A0.shape[1]
    H, C = a_s_up.shape
    wup = W_up.reshape(F, H * C)
    wend = W_end.reshape(2 * F, H * C)

    xu = _sc_unpool(X, idx0.astype(jnp.int32), Nn).reshape(B, Nn, F)

    neg_mask = pl.pallas_call(
        _mask_body,
        grid=(B,),
        in_specs=[pl.BlockSpec((1, Nn, Nn), lambda b: (b, 0, 0))],
        out_specs=pl.BlockSpec((1, Nn, Nn), lambda b: (b, 0, 0)),
        out_shape=jax.ShapeDtypeStruct((B, Nn, Nn), jnp.bfloat16),
    )(A0)

    log2e = 1.4426950408889634
    eye = jnp.eye(H, dtype=jnp.float32)
    asn_up = jnp.concatenate(
        [(a_s_up[:, :, None] * eye[:, None, :]).reshape(H * C, H),
         (a_n_up[:, :, None] * eye[:, None, :]).reshape(H * C, H)],
        axis=1) * log2e
    asn_end = jnp.concatenate(
        [(a_s_end[:, :, None] * eye[:, None, :]).reshape(H * C, H),
         (a_n_end[:, :, None] * eye[:, None, :]).reshape(H * C, H)],
        axis=1) * log2e

    scale = (jnp.asarray(l_n) / 1).astype(jnp.float32).reshape(1)

    full = lambda *shape: pl.BlockSpec(shape, lambda b: (0,) * len(shape))
    out = pl.pallas_call(
        _tc_body,
        grid=(B,),
        in_specs=[
            pl.BlockSpec(memory_space=pltpu.SMEM),
            pl.BlockSpec((1, Nn, F), lambda b: (b, 0, 0)),
            pl.BlockSpec((1, Nn, Nn), lambda b: (b, 0, 0)),
            pl.BlockSpec((1, Nn, F), lambda b: (b, 0, 0)),
            pl.BlockSpec((1, Nn, F), lambda b: (b, 0, 0)),
            full(F, H * C),
            full(H * C, 2 * H),
            full(2 * F, H * C),
            full(H * C, 2 * H),
        ],
        out_specs=pl.BlockSpec((1, Nn, F), lambda b: (b, 0, 0)),
        out_shape=jax.ShapeDtypeStruct((B, Nn, F), jnp.float32),
        compiler_params=pltpu.CompilerParams(
            dimension_semantics=("arbitrary",),
            vmem_limit_bytes=100 * 1024 * 1024,
        ),
    )(scale, xu, neg_mask, down0, orig_X, wup, asn_up, wend, asn_end)
    return out


# speedup vs baseline: 1.0711x; 1.0711x over previous
import functools

import jax
import jax.numpy as jnp
from jax.experimental import pallas as pl
from jax.experimental.pallas import tpu as pltpu
from jax.experimental.pallas import tpu_sc as plsc


def _elu(x):
    return jnp.where(x > 0, x, jnp.exp(jnp.minimum(x, 0.0)) - 1.0)


def _gat_block(feats, asn, neg_mask, H, C):
    N = feats.shape[0]
    acc = jnp.zeros((N, C), jnp.float32)
    inv_h = 1.0 / H
    feats_b = feats.astype(jnp.bfloat16)
    esen = jnp.dot(feats, asn, preferred_element_type=jnp.float32)
    enT = jnp.transpose(esen[:, H:])
    for h in range(H):
        t = esen[:, h:h + 1] + enT[h:h + 1, :]
        p = jnp.exp2(jnp.maximum(t, 0.2 * t) + neg_mask)
        pb = p.astype(jnp.bfloat16)
        s = jnp.sum(p, axis=1, keepdims=True)
        acc = acc + jnp.dot(pb, feats_b[:, h * C:(h + 1) * C],
                            preferred_element_type=jnp.float32) * (inv_h / s)
    return acc


def _tc_body(scale_ref, xu_ref, a_ref, down_ref, orig_ref,
             wup_ref, asn_up_ref, wend_ref, asn_end_ref, out_ref):
    F = xu_ref.shape[2]
    Nn = a_ref.shape[1]
    H = asn_up_ref.shape[1] // 2
    C = wup_ref.shape[1] // H

    a = a_ref[0]
    ri = jax.lax.broadcasted_iota(jnp.int32, (Nn, Nn), 0)
    ci = jax.lax.broadcasted_iota(jnp.int32, (Nn, Nn), 1)
    edge = jnp.logical_or(a > 0.5, ri == ci)
    neg_mask = jnp.where(edge, 0.0, -1e9).astype(jnp.float32)

    feats1 = jnp.dot(xu_ref[0], wup_ref[...],
                     preferred_element_type=jnp.float32)
    x1 = _elu(_gat_block(feats1, asn_up_ref[...], neg_mask, H, C))
    x1 = x1 + down_ref[0]

    feats2 = (jnp.dot(x1, wend_ref[:F, :], preferred_element_type=jnp.float32)
              + jnp.dot(orig_ref[0], wend_ref[F:, :],
                        preferred_element_type=jnp.float32))
    out_ref[0] = _elu(_gat_block(feats2, asn_end_ref[...], neg_mask,
                                 H, C)) * scale_ref[0]


def _sc_unpool(X, idx0, Nn):
    B, No, F = X.shape
    NC, NS = 2, 16
    bpc = B // NC
    rows_w = No // NS
    zrows = bpc * Nn // NS
    mesh = plsc.VectorSubcoreMesh(core_axis_name="c", subcore_axis_name="s")

    @functools.partial(
        pl.kernel, mesh=mesh,
        out_type=jax.ShapeDtypeStruct((B * Nn, F), jnp.float32),
        scratch_types=[
            [pltpu.VMEM((rows_w, F), jnp.float32)] * bpc,
            [pltpu.VMEM((rows_w,), jnp.int32)] * bpc,
            pltpu.VMEM((zrows, F), jnp.float32),
            pltpu.VMEM_SHARED((bpc * Nn, F), jnp.float32),
            [pltpu.SemaphoreType.DMA] * (2 * bpc),
        ])
    def k(x_hbm, idx_hbm, out_hbm, rows_v, idx_v, zbuf, acc_sh, sems):
        c = jax.lax.axis_index("c")
        s = jax.lax.axis_index("s")

        cps = []
        for lb in range(bpc):
            b = c * bpc + lb
            cps.append((
                pltpu.async_copy(idx_hbm.at[b, pl.ds(s * rows_w, rows_w)],
                                 idx_v[lb], sems[2 * lb]),
                pltpu.async_copy(x_hbm.at[b, pl.ds(s * rows_w, rows_w)],
                                 rows_v[lb], sems[2 * lb + 1]),
            ))

        def zero_row(r, carry):
            for kk in range(F // 16):
                zbuf[r, pl.ds(kk * 16, 16)] = jnp.zeros((16,), jnp.float32)
            return carry

        jax.lax.fori_loop(0, zrows, zero_row, 0)
        pltpu.sync_copy(zbuf, acc_sh.at[pl.ds(s * zrows, zrows)])
        plsc.subcore_barrier()

        for lb in range(bpc):
            cp_i, cp_r = cps[lb]
            cp_i.wait()
            if lb:
                for kk in range(rows_w // 16):
                    idx_v[lb][pl.ds(kk * 16, 16)] = (
                        idx_v[lb][pl.ds(kk * 16, 16)] + lb * Nn)
            cp_r.wait()
            pltpu.sync_copy(rows_v[lb], acc_sh.at[idx_v[lb]], add=True)
        plsc.subcore_barrier()

        base = c * (bpc * Nn) + s * zrows
        pltpu.sync_copy(acc_sh.at[pl.ds(s * zrows, zrows)],
                        out_hbm.at[pl.ds(base, zrows)])

    return k(X, idx0)


def kernel(X, orig_X, l_n, idx0, A0, down0, action, W_up, a_s_up, a_n_up,
           W_end, a_s_end, a_n_end):
    B, No, F = X.shape
    Nn = A0.shape[1]
    H, C = a_s_up.shape
    wup = W_up.reshape(F, H * C)
    wend = W_end.reshape(2 * F, H * C)

    xu = _sc_unpool(X, idx0.astype(jnp.int32), Nn).reshape(B, Nn, F)

    log2e = 1.4426950408889634
    eye = jnp.eye(H, dtype=jnp.float32)
    asn_up = jnp.concatenate(
        [(a_s_up[:, :, None] * eye[:, None, :]).reshape(H * C, H),
         (a_n_up[:, :, None] * eye[:, None, :]).reshape(H * C, H)],
        axis=1) * log2e
    asn_end = jnp.concatenate(
        [(a_s_end[:, :, None] * eye[:, None, :]).reshape(H * C, H),
         (a_n_end[:, :, None] * eye[:, None, :]).reshape(H * C, H)],
        axis=1) * log2e

    scale = (jnp.asarray(l_n) / 1).astype(jnp.float32).reshape(1)

    full = lambda *shape: pl.BlockSpec(shape, lambda b: (0,) * len(shape))
    out = pl.pallas_call(
        _tc_body,
        grid=(B,),
        in_specs=[
            pl.BlockSpec(memory_space=pltpu.SMEM),
            pl.BlockSpec((1, Nn, F), lambda b: (b, 0, 0)),
            pl.BlockSpec((1, Nn, Nn), lambda b: (b, 0, 0)),
            pl.BlockSpec((1, Nn, F), lambda b: (b, 0, 0)),
            pl.BlockSpec((1, Nn, F), lambda b: (b, 0, 0)),
            full(F, H * C),
            full(H * C, 2 * H),
            full(2 * F, H * C),
            full(H * C, 2 * H),
        ],
        out_specs=pl.BlockSpec((1, Nn, F), lambda b: (b, 0, 0)),
        out_shape=jax.ShapeDtypeStruct((B, Nn, F), jnp.float32),
        compiler_params=pltpu.CompilerParams(
            dimension_semantics=("arbitrary",),
            vmem_limit_bytes=100 * 1024 * 1024,
        ),
    )(scale, xu, A0, down0, orig_X, wup, asn_up, wend, asn_end)
    return out


# speedup vs baseline: 1.0716x; 1.0005x over previous
import functools

import jax
import jax.numpy as jnp
from jax.experimental import pallas as pl
from jax.experimental.pallas import tpu as pltpu
from jax.experimental.pallas import tpu_sc as plsc


def _elu(x):
    return jnp.where(x > 0, x, jnp.exp(jnp.minimum(x, 0.0)) - 1.0)


def _gat_block(feats, asn, neg_mask, H, C):
    N = feats.shape[0]
    acc = jnp.zeros((N, C), jnp.float32)
    inv_h = 1.0 / H
    feats_b = feats.astype(jnp.bfloat16)
    esen = jnp.dot(feats, asn, preferred_element_type=jnp.float32)
    enT = jnp.transpose(esen[:, H:])
    for h in range(H):
        t = esen[:, h:h + 1] + enT[h:h + 1, :]
        p = jnp.exp2(jnp.maximum(t, 0.2 * t) + neg_mask)
        pb = p.astype(jnp.bfloat16)
        s = jnp.sum(p, axis=1, keepdims=True)
        acc = acc + jnp.dot(pb, feats_b[:, h * C:(h + 1) * C],
                            preferred_element_type=jnp.float32) * (inv_h / s)
    return acc


def _tc_body(scale_ref, xu_ref, a_ref, down_ref, orig_ref,
             wup_ref, asn_up_ref, wend_ref, asn_end_ref, out_ref):
    F = xu_ref.shape[2]
    Nn = a_ref.shape[1]
    H = asn_up_ref.shape[1] // 2
    C = wup_ref.shape[1] // H

    a = a_ref[0]
    ri = jax.lax.broadcasted_iota(jnp.int32, (Nn, Nn), 0)
    ci = jax.lax.broadcasted_iota(jnp.int32, (Nn, Nn), 1)
    edge = jnp.logical_or(a > 0.5, ri == ci)
    neg_mask = jnp.where(edge, 0.0, -1e9).astype(jnp.float32)

    feats1 = jnp.dot(xu_ref[0], wup_ref[...],
                     preferred_element_type=jnp.float32)
    x1 = _elu(_gat_block(feats1, asn_up_ref[...], neg_mask, H, C))
    x1 = x1 + down_ref[0]

    feats2 = (jnp.dot(x1, wend_ref[:F, :], preferred_element_type=jnp.float32)
              + jnp.dot(orig_ref[0], wend_ref[F:, :],
                        preferred_element_type=jnp.float32))
    out_ref[0] = _elu(_gat_block(feats2, asn_end_ref[...], neg_mask,
                                 H, C)) * scale_ref[0]


def _sc_unpool(X, idx0, Nn):
    B, No, F = X.shape
    NC, NS = 2, 16
    bpc = B // NC
    rows_w = No // NS
    zrows = bpc * Nn // NS
    mesh = plsc.VectorSubcoreMesh(core_axis_name="c", subcore_axis_name="s")

    @functools.partial(
        pl.kernel, mesh=mesh,
        out_type=jax.ShapeDtypeStruct((B * Nn, F), jnp.float32),
        scratch_types=[
            [pltpu.VMEM((rows_w, F), jnp.float32)] * bpc,
            [pltpu.VMEM((rows_w,), jnp.int32)] * bpc,
            pltpu.VMEM((zrows, F), jnp.float32),
            pltpu.VMEM_SHARED((bpc * Nn, F), jnp.float32),
            [pltpu.SemaphoreType.DMA] * (2 * bpc),
        ])
    def k(x_hbm, idx_hbm, out_hbm, rows_v, idx_v, zbuf, acc_sh, sems):
        c = jax.lax.axis_index("c")
        s = jax.lax.axis_index("s")

        cps = []
        for lb in range(bpc):
            b = c * bpc + lb
            cps.append((
                pltpu.async_copy(idx_hbm.at[b, pl.ds(s * rows_w, rows_w)],
                                 idx_v[lb], sems[2 * lb]),
                pltpu.async_copy(x_hbm.at[b, pl.ds(s * rows_w, rows_w)],
                                 rows_v[lb], sems[2 * lb + 1]),
            ))

        def zero_row(r, carry):
            for kk in range(F // 16):
                zbuf[r, pl.ds(kk * 16, 16)] = jnp.zeros((16,), jnp.float32)
            return carry

        jax.lax.fori_loop(0, zrows, zero_row, 0)
        pltpu.sync_copy(zbuf, acc_sh.at[pl.ds(s * zrows, zrows)])
        plsc.subcore_barrier()

        for lb in range(bpc):
            cp_i, cp_r = cps[lb]
            cp_i.wait()
            if lb:
                for kk in range(rows_w // 16):
                    idx_v[lb][pl.ds(kk * 16, 16)] = (
                        idx_v[lb][pl.ds(kk * 16, 16)] + lb * Nn)
            cp_r.wait()
            pltpu.sync_copy(rows_v[lb], acc_sh.at[idx_v[lb]], add=True)
        plsc.subcore_barrier()

        base = c * (bpc * Nn) + s * zrows
        pltpu.sync_copy(acc_sh.at[pl.ds(s * zrows, zrows)],
                        out_hbm.at[pl.ds(base, zrows)])

    return k(X, idx0)


def kernel(X, orig_X, l_n, idx0, A0, down0, action, W_up, a_s_up, a_n_up,
           W_end, a_s_end, a_n_end):
    B, No, F = X.shape
    Nn = A0.shape[1]
    H, C = a_s_up.shape
    wup = W_up.reshape(F, H * C)
    wend = W_end.reshape(2 * F, H * C)

    xu = _sc_unpool(X, idx0.astype(jnp.int32), Nn).reshape(B, Nn, F)

    log2e = 1.4426950408889634
    eye = jnp.eye(H, dtype=jnp.float32)
    asn_up = jnp.concatenate(
        [(a_s_up[:, :, None] * eye[:, None, :]).reshape(H * C, H),
         (a_n_up[:, :, None] * eye[:, None, :]).reshape(H * C, H)],
        axis=1) * log2e
    asn_end = jnp.concatenate(
        [(a_s_end[:, :, None] * eye[:, None, :]).reshape(H * C, H),
         (a_n_end[:, :, None] * eye[:, None, :]).reshape(H * C, H)],
        axis=1) * log2e

    scale = (jnp.asarray(l_n) / 1).astype(jnp.float32).reshape(1)

    full = lambda *shape: pl.BlockSpec(shape, lambda b: (0,) * len(shape))
    out = pl.pallas_call(
        _tc_body,
        grid=(B,),
        in_specs=[
            pl.BlockSpec(memory_space=pltpu.SMEM),
            pl.BlockSpec((1, Nn, F), lambda b: (b, 0, 0)),
            pl.BlockSpec((1, Nn, Nn), lambda b: (b, 0, 0)),
            pl.BlockSpec((1, Nn, F), lambda b: (b, 0, 0)),
            pl.BlockSpec((1, Nn, F), lambda b: (b, 0, 0)),
            full(F, H * C),
            full(H * C, 2 * H),
            full(2 * F, H * C),
            full(H * C, 2 * H),
        ],
        out_specs=pl.BlockSpec((1, Nn, F), lambda b: (b, 0, 0)),
        out_shape=jax.ShapeDtypeStruct((B, Nn, F), jnp.float32),
        compiler_params=pltpu.CompilerParams(
            dimension_semantics=("parallel",),
            vmem_limit_bytes=100 * 1024 * 1024,
        ),
    )(scale, xu, A0, down0, orig_X, wup, asn_up, wend, asn_end)
    return out
